# Initial kernel scaffold; baseline (speedup 1.0000x reference)
import functools

import jax
import jax.numpy as jnp
from jax import lax
from jax.experimental import pallas as pl
from jax.experimental.pallas import tpu as pltpu
from jax.experimental.pallas import tpu_sc as plsc

N = 10000
E = 320000
D = 128

NC = 2
NS = 16
N_PAD = 10240
DEG_TILE = N_PAD // NS

CHUNK = 80
DEG_CHUNKS = (E // NS) // CHUNK
AGG_CHUNKS = (E // (NC * NS)) // CHUNK

ROWS_TILE = N // NS
ZROWS = 125

_mesh = plsc.VectorSubcoreMesh(core_axis_name="c", subcore_axis_name="s")


@functools.partial(
    pl.kernel,
    out_type=jax.ShapeDtypeStruct((2, N_PAD), jnp.float32),
    mesh=_mesh,
    scratch_types=[
        pltpu.VMEM_SHARED((N_PAD,), jnp.float32),
        pltpu.VMEM((CHUNK,), jnp.int32),
        pltpu.VMEM((CHUNK,), jnp.float32),
        pltpu.VMEM((DEG_TILE,), jnp.float32),
    ],
)
def _degree_kernel(edges_hbm, out_hbm, hist, idx_v, ones_v, buf_v):
    c = lax.axis_index("c")
    s = lax.axis_index("s")

    def fill_ones(i, _):
        ones_v[pl.ds(i * 16, 16)] = jnp.ones((16,), jnp.float32)
        return 0

    lax.fori_loop(0, CHUNK // 16, fill_ones, 0)

    def fill_zero(i, _):
        buf_v[pl.ds(i * 16, 16)] = jnp.zeros((16,), jnp.float32)
        return 0

    lax.fori_loop(0, DEG_TILE // 16, fill_zero, 0)
    pltpu.sync_copy(buf_v, hist.at[pl.ds(s * DEG_TILE, DEG_TILE)])
    plsc.subcore_barrier()

    base = s * (E // NS)

    def body(j, _):
        pltpu.sync_copy(edges_hbm.at[c, pl.ds(base + j * CHUNK, CHUNK)], idx_v)
        pltpu.sync_copy(ones_v, hist.at[idx_v], add=True)
        return 0

    lax.fori_loop(0, DEG_CHUNKS, body, 0)
    plsc.subcore_barrier()

    pltpu.sync_copy(hist.at[pl.ds(s * DEG_TILE, DEG_TILE)], buf_v)
    pltpu.sync_copy(buf_v, out_hbm.at[c, pl.ds(s * DEG_TILE, DEG_TILE)])


@functools.partial(
    pl.kernel,
    out_type=jax.ShapeDtypeStruct((NC, N, D), jnp.float32),
    mesh=_mesh,
    scratch_types=[
        pltpu.VMEM_SHARED((N, D), jnp.float32),
        pltpu.VMEM((CHUNK,), jnp.int32),
        pltpu.VMEM((CHUNK,), jnp.int32),
        pltpu.VMEM((CHUNK, D), jnp.float32),
        pltpu.VMEM((ZROWS, D), jnp.float32),
        pltpu.SemaphoreType.DMA,
    ],
)
def _agg_kernel(u_hbm, edges_hbm, out_hbm, acc, sidx, didx, rows, zbuf, sem):
    c = lax.axis_index("c")
    s = lax.axis_index("s")

    def fill_zero(r, _):
        for k in range(D // 16):
            zbuf[r, pl.ds(k * 16, 16)] = jnp.zeros((16,), jnp.float32)
        return 0

    lax.fori_loop(0, ZROWS, fill_zero, 0)
    for k in range(ROWS_TILE // ZROWS):
        pltpu.sync_copy(zbuf, acc.at[pl.ds(s * ROWS_TILE + k * ZROWS, ZROWS)])
    plsc.subcore_barrier()

    ebase = (c * NS + s) * (E // (NC * NS))

    def body(j, _):
        off = ebase + j * CHUNK
        pltpu.sync_copy(edges_hbm.at[0, pl.ds(off, CHUNK)], sidx)
        pltpu.sync_copy(edges_hbm.at[1, pl.ds(off, CHUNK)], didx)
        pltpu.async_copy(u_hbm.at[sidx], rows, sem).wait()
        pltpu.sync_copy(rows, acc.at[didx], add=True)
        return 0

    lax.fori_loop(0, AGG_CHUNKS, body, 0)
    plsc.subcore_barrier()

    for k in range(ROWS_TILE // ZROWS):
        r0 = s * ROWS_TILE + k * ZROWS
        pltpu.sync_copy(acc.at[pl.ds(r0, ZROWS)], zbuf)
        pltpu.sync_copy(zbuf, out_hbm.at[c, pl.ds(r0, ZROWS)])


_R = 1000


def _rs(deg_blk):
    return lax.rsqrt(jnp.maximum(deg_blk, 1.0))


def _mm_in_body(x_ref, w_ref, dego_ref, o_ref):
    o_ref[...] = jnp.dot(
        x_ref[...], w_ref[...], preferred_element_type=jnp.float32
    ) * _rs(dego_ref[...])


def _mm_in(x, w, dego):
    return pl.pallas_call(
        _mm_in_body,
        grid=(N // _R,),
        in_specs=[
            pl.BlockSpec((_R, D), lambda i: (i, 0)),
            pl.BlockSpec((D, D), lambda i: (0, 0)),
            pl.BlockSpec((_R, 1), lambda i: (i, 0)),
        ],
        out_specs=pl.BlockSpec((_R, D), lambda i: (i, 0)),
        out_shape=jax.ShapeDtypeStruct((N, D), jnp.float32),
    )(x, w, dego)


def _mid_body(acc_ref, degi_ref, dego_ref, b_ref, w_ref, o_ref):
    a = acc_ref[0] + acc_ref[1]
    rst = jnp.maximum(a * _rs(degi_ref[...]) + b_ref[...], 0.0)
    o_ref[...] = jnp.dot(
        rst, w_ref[...], preferred_element_type=jnp.float32
    ) * _rs(dego_ref[...])


def _mid(acc, degi, dego, b, w):
    return pl.pallas_call(
        _mid_body,
        grid=(N // _R,),
        in_specs=[
            pl.BlockSpec((NC, _R, D), lambda i: (0, i, 0)),
            pl.BlockSpec((_R, 1), lambda i: (i, 0)),
            pl.BlockSpec((_R, 1), lambda i: (i, 0)),
            pl.BlockSpec((1, D), lambda i: (0, 0)),
            pl.BlockSpec((D, D), lambda i: (0, 0)),
        ],
        out_specs=pl.BlockSpec((_R, D), lambda i: (i, 0)),
        out_shape=jax.ShapeDtypeStruct((N, D), jnp.float32),
    )(acc, degi, dego, b, w)


def _final_body(acc_ref, degi_ref, b_ref, o_ref):
    a = acc_ref[0] + acc_ref[1]
    o_ref[...] = a * _rs(degi_ref[...]) + b_ref[...]


def _final(acc, degi, b):
    return pl.pallas_call(
        _final_body,
        grid=(N // _R,),
        in_specs=[
            pl.BlockSpec((NC, _R, D), lambda i: (0, i, 0)),
            pl.BlockSpec((_R, 1), lambda i: (i, 0)),
            pl.BlockSpec((1, D), lambda i: (0, 0)),
        ],
        out_specs=pl.BlockSpec((_R, D), lambda i: (i, 0)),
        out_shape=jax.ShapeDtypeStruct((N, D), jnp.float32),
    )(acc, degi, b)


def kernel(features, edge_index, W1, b1, W2, b2):
    deg = _degree_kernel(edge_index)
    dego = deg[0, :N].reshape(N, 1)
    degi = deg[1, :N].reshape(N, 1)
    b1r = b1.reshape(1, D)
    b2r = b2.reshape(1, D)

    u1 = _mm_in(features, W1, dego)
    acc1 = _agg_kernel(u1, edge_index)
    u2 = _mid(acc1, degi, dego, b1r, W2)
    acc2 = _agg_kernel(u2, edge_index)
    out = _final(acc2, degi, b2r)
    return out


# speedup vs baseline: 4.9315x; 4.9315x over previous
import functools

import jax
import jax.numpy as jnp
from jax import lax
from jax.experimental import pallas as pl
from jax.experimental.pallas import tpu as pltpu
from jax.experimental.pallas import tpu_sc as plsc

N = 10000
E = 320000
D = 128

NC = 2
NS = 16
N_PAD = 10240
DEG_TILE = N_PAD // NS

CHUNK = 80
DEG_CHUNKS = (E // NS) // CHUNK
AGG_CHUNKS = (E // (NC * NS)) // CHUNK

ROWS_TILE = N_PAD // NS
ZROWS = 128

_mesh = plsc.VectorSubcoreMesh(core_axis_name="c", subcore_axis_name="s")


@functools.partial(
    pl.kernel,
    out_type=jax.ShapeDtypeStruct((2 * N_PAD,), jnp.float32),
    mesh=_mesh,
    scratch_types=[
        pltpu.VMEM_SHARED((N_PAD,), jnp.float32),
        pltpu.VMEM((CHUNK,), jnp.int32),
        pltpu.VMEM((CHUNK,), jnp.float32),
        pltpu.VMEM((DEG_TILE,), jnp.float32),
    ],
)
def _degree_kernel(src_hbm, dst_hbm, out_hbm, hist, idx_v, ones_v, buf_v):
    c = lax.axis_index("c")
    s = lax.axis_index("s")

    def fill_ones(i, _):
        ones_v[pl.ds(i * 16, 16)] = jnp.ones((16,), jnp.float32)
        return 0

    lax.fori_loop(0, CHUNK // 16, fill_ones, 0)

    def fill_zero(i, _):
        buf_v[pl.ds(i * 16, 16)] = jnp.zeros((16,), jnp.float32)
        return 0

    lax.fori_loop(0, DEG_TILE // 16, fill_zero, 0)
    pltpu.sync_copy(buf_v, hist.at[pl.ds(s * DEG_TILE, DEG_TILE)])
    plsc.subcore_barrier()

    base = s * (E // NS)

    def body(j, _):
        off = base + j * CHUNK

        @pl.when(c == 0)
        def _():
            pltpu.sync_copy(src_hbm.at[pl.ds(off, CHUNK)], idx_v)

        @pl.when(c == 1)
        def _():
            pltpu.sync_copy(dst_hbm.at[pl.ds(off, CHUNK)], idx_v)

        pltpu.sync_copy(ones_v, hist.at[idx_v], add=True)
        return 0

    lax.fori_loop(0, DEG_CHUNKS, body, 0)
    plsc.subcore_barrier()

    pltpu.sync_copy(hist.at[pl.ds(s * DEG_TILE, DEG_TILE)], buf_v)
    pltpu.sync_copy(buf_v, out_hbm.at[pl.ds(c * N_PAD + s * DEG_TILE, DEG_TILE)])


@functools.partial(
    pl.kernel,
    out_type=jax.ShapeDtypeStruct((NC, N_PAD, D), jnp.float32),
    mesh=_mesh,
    scratch_types=[
        pltpu.VMEM_SHARED((N_PAD, D), jnp.float32),
        pltpu.VMEM((CHUNK,), jnp.int32),
        pltpu.VMEM((CHUNK,), jnp.int32),
        pltpu.VMEM((CHUNK, D), jnp.float32),
        pltpu.VMEM((ZROWS, D), jnp.float32),
        pltpu.SemaphoreType.DMA,
    ],
)
def _agg_kernel(u_hbm, src_hbm, dst_hbm, out_hbm, acc, sidx, didx, rows, zbuf, sem):
    c = lax.axis_index("c")
    s = lax.axis_index("s")

    def fill_zero(r, _):
        for k in range(D // 16):
            zbuf[r, pl.ds(k * 16, 16)] = jnp.zeros((16,), jnp.float32)
        return 0

    lax.fori_loop(0, ZROWS, fill_zero, 0)
    for k in range(ROWS_TILE // ZROWS):
        pltpu.sync_copy(zbuf, acc.at[pl.ds(s * ROWS_TILE + k * ZROWS, ZROWS)])
    plsc.subcore_barrier()

    ebase = (c * NS + s) * (E // (NC * NS))

    def body(j, _):
        off = ebase + j * CHUNK
        pltpu.sync_copy(src_hbm.at[pl.ds(off, CHUNK)], sidx)
        pltpu.sync_copy(dst_hbm.at[pl.ds(off, CHUNK)], didx)
        pltpu.async_copy(u_hbm.at[sidx], rows, sem).wait()
        pltpu.sync_copy(rows, acc.at[didx], add=True)
        return 0

    lax.fori_loop(0, AGG_CHUNKS, body, 0)
    plsc.subcore_barrier()

    for k in range(ROWS_TILE // ZROWS):
        r0 = s * ROWS_TILE + k * ZROWS
        pltpu.sync_copy(acc.at[pl.ds(r0, ZROWS)], zbuf)
        pltpu.sync_copy(zbuf, out_hbm.at[c, pl.ds(r0, ZROWS)])


_R = 1000


def _rs(deg_blk):
    return lax.rsqrt(jnp.maximum(deg_blk, 1.0))


def _mm_in_body(x_ref, w_ref, dego_ref, o_ref):
    o_ref[...] = jnp.dot(
        x_ref[...], w_ref[...], preferred_element_type=jnp.float32
    ) * _rs(dego_ref[...])


def _mm_in(x, w, dego):
    return pl.pallas_call(
        _mm_in_body,
        grid=(N // _R,),
        in_specs=[
            pl.BlockSpec((_R, D), lambda i: (i, 0)),
            pl.BlockSpec((D, D), lambda i: (0, 0)),
            pl.BlockSpec((_R, 1), lambda i: (i, 0)),
        ],
        out_specs=pl.BlockSpec((_R, D), lambda i: (i, 0)),
        out_shape=jax.ShapeDtypeStruct((N, D), jnp.float32),
    )(x, w, dego)


def _mid_body(acc_ref, degi_ref, dego_ref, b_ref, w_ref, o_ref):
    a = acc_ref[0] + acc_ref[1]
    rst = jnp.maximum(a * _rs(degi_ref[...]) + b_ref[...], 0.0)
    o_ref[...] = jnp.dot(
        rst, w_ref[...], preferred_element_type=jnp.float32
    ) * _rs(dego_ref[...])


def _mid(acc, degi, dego, b, w):
    return pl.pallas_call(
        _mid_body,
        grid=(N // _R,),
        in_specs=[
            pl.BlockSpec((NC, _R, D), lambda i: (0, i, 0)),
            pl.BlockSpec((_R, 1), lambda i: (i, 0)),
            pl.BlockSpec((_R, 1), lambda i: (i, 0)),
            pl.BlockSpec((1, D), lambda i: (0, 0)),
            pl.BlockSpec((D, D), lambda i: (0, 0)),
        ],
        out_specs=pl.BlockSpec((_R, D), lambda i: (i, 0)),
        out_shape=jax.ShapeDtypeStruct((N, D), jnp.float32),
    )(acc, degi, dego, b, w)


def _final_body(acc_ref, degi_ref, b_ref, o_ref):
    a = acc_ref[0] + acc_ref[1]
    o_ref[...] = a * _rs(degi_ref[...]) + b_ref[...]


def _final(acc, degi, b):
    return pl.pallas_call(
        _final_body,
        grid=(N // _R,),
        in_specs=[
            pl.BlockSpec((NC, _R, D), lambda i: (0, i, 0)),
            pl.BlockSpec((_R, 1), lambda i: (i, 0)),
            pl.BlockSpec((1, D), lambda i: (0, 0)),
        ],
        out_specs=pl.BlockSpec((_R, D), lambda i: (i, 0)),
        out_shape=jax.ShapeDtypeStruct((N, D), jnp.float32),
    )(acc, degi, b)


def kernel(features, edge_index, W1, b1, W2, b2):
    src = edge_index[0]
    dst = edge_index[1]
    deg = _degree_kernel(src, dst)
    dego = deg[:N].reshape(N, 1)
    degi = deg[N_PAD:N_PAD + N].reshape(N, 1)
    b1r = b1.reshape(1, D)
    b2r = b2.reshape(1, D)

    u1 = _mm_in(features, W1, dego)
    acc1 = _agg_kernel(u1, src, dst)
    u2 = _mid(acc1, degi, dego, b1r, W2)
    acc2 = _agg_kernel(u2, src, dst)
    out = _final(acc2, degi, b2r)
    return out


# speedup vs baseline: 14.6836x; 2.9775x over previous
import functools

import jax
import jax.numpy as jnp
from jax import lax
from jax.experimental import pallas as pl
from jax.experimental.pallas import tpu as pltpu
from jax.experimental.pallas import tpu_sc as plsc

N = 10000
E = 320000
D = 128

NC = 2
NS = 16
N_PAD = 10240
DEG_TILE = N_PAD // NS

DCHUNK = 125
DPT = (E // NS) // DCHUNK
CHUNK = 80
CPT = (E // (NC * NS)) // CHUNK
RING = 3
RI = 6
DEG_Q = 8

ROWS_TILE = N_PAD // NS
ZROWS = 64

_mesh = plsc.VectorSubcoreMesh(core_axis_name="c", subcore_axis_name="s")


@functools.partial(
    pl.kernel,
    out_type=jax.ShapeDtypeStruct((2 * N_PAD,), jnp.float32),
    mesh=_mesh,
    scratch_types=[
        pltpu.VMEM_SHARED((N_PAD,), jnp.float32),
        pltpu.VMEM((DPT, DCHUNK), jnp.int32),
        pltpu.VMEM((128,), jnp.float32),
        pltpu.VMEM((DEG_TILE,), jnp.float32),
        pltpu.SemaphoreType.DMA,
    ],
)
def _degree_kernel(src_hbm, dst_hbm, out_hbm, hist, idx_v, ones_v, buf_v, sem):
    c = lax.axis_index("c")
    s = lax.axis_index("s")

    def fill_ones(i, _):
        ones_v[pl.ds(i * 16, 16)] = jnp.ones((16,), jnp.float32)
        return 0

    lax.fori_loop(0, 8, fill_ones, 0)

    def fill_zero(i, _):
        buf_v[pl.ds(i * 16, 16)] = jnp.zeros((16,), jnp.float32)
        return 0

    lax.fori_loop(0, DEG_TILE // 16, fill_zero, 0)
    pltpu.sync_copy(buf_v, hist.at[pl.ds(s * DEG_TILE, DEG_TILE)])

    @pl.when(c == 0)
    def _():
        pltpu.sync_copy(src_hbm.at[s], idx_v)

    @pl.when(c == 1)
    def _():
        pltpu.sync_copy(dst_hbm.at[s], idx_v)

    plsc.subcore_barrier()

    ones_row = ones_v.at[pl.ds(0, DCHUNK)]

    def body(i, _):
        pltpu.make_async_copy(ones_row, hist.at[idx_v.at[i]], sem).start(add=True)

        @pl.when(i >= DEG_Q)
        def _():
            pltpu.make_async_copy(ones_row, hist.at[idx_v.at[i]], sem).wait()

        return 0

    lax.fori_loop(0, DPT, body, 0)
    for _ in range(DEG_Q):
        pltpu.make_async_copy(ones_row, hist.at[idx_v.at[0]], sem).wait()

    plsc.subcore_barrier()
    pltpu.sync_copy(hist.at[pl.ds(s * DEG_TILE, DEG_TILE)], buf_v)
    pltpu.sync_copy(buf_v, out_hbm.at[pl.ds(c * N_PAD + s * DEG_TILE, DEG_TILE)])


@functools.partial(
    pl.kernel,
    out_type=jax.ShapeDtypeStruct((NC, N_PAD, D), jnp.float32),
    mesh=_mesh,
    scratch_types=[
        pltpu.VMEM_SHARED((N_PAD, D), jnp.float32),
        pltpu.VMEM((RI, CHUNK), jnp.int32),
        pltpu.VMEM((RI, CHUNK), jnp.int32),
        pltpu.VMEM((RING, CHUNK, D), jnp.float32),
        pltpu.SemaphoreType.DMA((RING,)),
        pltpu.SemaphoreType.DMA((RING,)),
        pltpu.SemaphoreType.DMA((RI,)),
    ],
)
def _agg_kernel(u_hbm, src_hbm, dst_hbm, out_hbm, acc, sidx, didx, rows,
                gsem, ssem, isem):
    c = lax.axis_index("c")
    s = lax.axis_index("s")
    ebase = (c * NS + s) * (E // (NC * NS))

    zbuf = rows.at[0]

    def fill_zero(r, _):
        for k in range(D // 16):
            rows[0, r, pl.ds(k * 16, 16)] = jnp.zeros((16,), jnp.float32)
        return 0

    lax.fori_loop(0, CHUNK, fill_zero, 0)
    for k in range(ROWS_TILE // CHUNK):
        pltpu.sync_copy(zbuf, acc.at[pl.ds(s * ROWS_TILE + k * CHUNK, CHUNK)])
    plsc.subcore_barrier()

    def fire_idx(i, q):
        off = ebase + i * CHUNK
        pltpu.make_async_copy(src_hbm.at[pl.ds(off, CHUNK)], sidx.at[q],
                              isem.at[q]).start()
        pltpu.make_async_copy(dst_hbm.at[pl.ds(off, CHUNK)], didx.at[q],
                              isem.at[q]).start()

    def wait_idx(q):
        pltpu.make_async_copy(src_hbm.at[pl.ds(0, CHUNK)], sidx.at[q],
                              isem.at[q]).wait()
        pltpu.make_async_copy(dst_hbm.at[pl.ds(0, CHUNK)], didx.at[q],
                              isem.at[q]).wait()

    def fire_gather(q, b):
        pltpu.make_async_copy(u_hbm.at[sidx.at[q]], rows.at[b], gsem.at[b]).start()

    def wait_gather(q, b):
        pltpu.make_async_copy(u_hbm.at[sidx.at[q]], rows.at[b], gsem.at[b]).wait()

    def fire_scatter(q, b):
        pltpu.make_async_copy(rows.at[b], acc.at[didx.at[q]], ssem.at[b]).start(add=True)

    def wait_scatter(q, b):
        pltpu.make_async_copy(rows.at[b], acc.at[didx.at[q]], ssem.at[b]).wait()

    fire_idx(0, 0)
    fire_idx(1, 1)
    for v in range(5):
        if v >= 3:
            wait_scatter((v - 3) % RI, v % RING)
        wait_idx(v % RI)
        fire_gather(v % RI, v % RING)
        fire_idx(v + 2, (v + 2) % RI)
        if v >= 2:
            wait_gather((v - 2) % RI, (v - 2) % RING)
            fire_scatter((v - 2) % RI, (v - 2) % RING)

    def outer(k, _):
        j0 = 5 + k * 6
        for v in range(6):
            i = j0 + v
            b = (5 + v) % RING
            q = (5 + v) % RI
            wait_scatter((2 + v) % RI, b)
            wait_idx(q)
            fire_gather(q, b)

            @pl.when(i + 2 < CPT)
            def _():
                fire_idx(i + 2, (1 + v) % RI)

            wait_gather((3 + v) % RI, v % RING)
            fire_scatter((3 + v) % RI, v % RING)
        return 0

    lax.fori_loop(0, (CPT - 5) // 6, outer, 0)

    for i in (CPT - 2, CPT - 1):
        wait_gather(i % RI, i % RING)
        fire_scatter(i % RI, i % RING)
    for i in range(CPT - RING, CPT):
        wait_scatter(i % RI, i % RING)

    plsc.subcore_barrier()

    for k in range(ROWS_TILE // CHUNK):
        r0 = s * ROWS_TILE + k * CHUNK
        pltpu.sync_copy(acc.at[pl.ds(r0, CHUNK)], zbuf)
        pltpu.sync_copy(zbuf, out_hbm.at[c, pl.ds(r0, CHUNK)])


_R = 1000


def _rs(deg_blk):
    return lax.rsqrt(jnp.maximum(deg_blk, 1.0))


def _mm_in_body(x_ref, w_ref, dego_ref, o_ref):
    o_ref[...] = jnp.dot(
        x_ref[...], w_ref[...], preferred_element_type=jnp.float32
    ) * _rs(dego_ref[...])


def _mm_in(x, w, dego):
    return pl.pallas_call(
        _mm_in_body,
        grid=(N // _R,),
        in_specs=[
            pl.BlockSpec((_R, D), lambda i: (i, 0)),
            pl.BlockSpec((D, D), lambda i: (0, 0)),
            pl.BlockSpec((_R, 1), lambda i: (i, 0)),
        ],
        out_specs=pl.BlockSpec((_R, D), lambda i: (i, 0)),
        out_shape=jax.ShapeDtypeStruct((N, D), jnp.float32),
    )(x, w, dego)


def _mid_body(acc_ref, degi_ref, dego_ref, b_ref, w_ref, o_ref):
    a = acc_ref[0] + acc_ref[1]
    rst = jnp.maximum(a * _rs(degi_ref[...]) + b_ref[...], 0.0)
    o_ref[...] = jnp.dot(
        rst, w_ref[...], preferred_element_type=jnp.float32
    ) * _rs(dego_ref[...])


def _mid(acc, degi, dego, b, w):
    return pl.pallas_call(
        _mid_body,
        grid=(N // _R,),
        in_specs=[
            pl.BlockSpec((NC, _R, D), lambda i: (0, i, 0)),
            pl.BlockSpec((_R, 1), lambda i: (i, 0)),
            pl.BlockSpec((_R, 1), lambda i: (i, 0)),
            pl.BlockSpec((1, D), lambda i: (0, 0)),
            pl.BlockSpec((D, D), lambda i: (0, 0)),
        ],
        out_specs=pl.BlockSpec((_R, D), lambda i: (i, 0)),
        out_shape=jax.ShapeDtypeStruct((N, D), jnp.float32),
    )(acc, degi, dego, b, w)


def _final_body(acc_ref, degi_ref, b_ref, o_ref):
    a = acc_ref[0] + acc_ref[1]
    o_ref[...] = a * _rs(degi_ref[...]) + b_ref[...]


def _final(acc, degi, b):
    return pl.pallas_call(
        _final_body,
        grid=(N // _R,),
        in_specs=[
            pl.BlockSpec((NC, _R, D), lambda i: (0, i, 0)),
            pl.BlockSpec((_R, 1), lambda i: (i, 0)),
            pl.BlockSpec((1, D), lambda i: (0, 0)),
        ],
        out_specs=pl.BlockSpec((_R, D), lambda i: (i, 0)),
        out_shape=jax.ShapeDtypeStruct((N, D), jnp.float32),
    )(acc, degi, b)


def kernel(features, edge_index, W1, b1, W2, b2):
    src = edge_index[0]
    dst = edge_index[1]
    src4 = src.reshape(NS, DPT, DCHUNK)
    dst4 = dst.reshape(NS, DPT, DCHUNK)

    deg = _degree_kernel(src4, dst4)
    dego = deg[:N].reshape(N, 1)
    degi = deg[N_PAD:N_PAD + N].reshape(N, 1)
    b1r = b1.reshape(1, D)
    b2r = b2.reshape(1, D)

    u1 = _mm_in(features, W1, dego)
    acc1 = _agg_kernel(u1, src, dst)
    u2 = _mid(acc1, degi, dego, b1r, W2)
    acc2 = _agg_kernel(u2, src, dst)
    out = _final(acc2, degi, b2r)
    return out
